# baseline (speedup 1.0000x reference)
import functools

import jax
import jax.numpy as jnp
from jax import lax
from jax.experimental import pallas as pl
from jax.experimental.pallas import tpu as pltpu
from jax.experimental.pallas import tpu_sc as plsc

VOCAB = 100000
SEQ_LEN = 2048
DIM = 768
BATCH = 4
N = 10000

NUM_CORES = 2
NUM_SUBCORES = 16
NW = NUM_CORES * NUM_SUBCORES
SCHUNK = 32
NCHUNK = SEQ_LEN // SCHUNK
CHUNKS_PER_W = NCHUNK // NW
LANES = 16
VECS_PER_ROW = DIM // LANES


def _positional_table():
    positions = jnp.arange(0, SEQ_LEN, dtype=jnp.float32)[:, None]
    den_even = jnp.power(float(N), 2.0 * jnp.arange(0, DIM, 2, dtype=jnp.float32) / DIM)
    den_odd = jnp.power(float(N), 2.0 * jnp.arange(1, DIM, 2, dtype=jnp.float32) / DIM)
    emb = jnp.zeros((SEQ_LEN, DIM), dtype=jnp.float32)
    emb = emb.at[:, 0::2].set(jnp.sin(positions / den_even))
    emb = emb.at[:, 1::2].set(jnp.cos(positions / den_odd))
    return emb


def _sc_body(table_hbm, x_hbm, pos_hbm, out_hbm, idx_v, pos_v, rows_v, sem):
    wid = lax.axis_index("s") * NUM_CORES + lax.axis_index("c")
    for k in range(CHUNKS_PER_W):
        c = wid * CHUNKS_PER_W + k
        s0 = c * SCHUNK
        pltpu.sync_copy(pos_hbm.at[pl.ds(s0, SCHUNK)], pos_v)
        for b in range(BATCH):
            base = b * SEQ_LEN + s0
            pltpu.sync_copy(x_hbm.at[pl.ds(base, SCHUNK)], idx_v)
            pltpu.async_copy(table_hbm.at[idx_v], rows_v, sem).wait()

            def row_add(r, _):
                def vec_add(j, _):
                    sl = pl.ds(j * LANES, LANES)
                    rows_v[r, sl] = rows_v[r, sl] + pos_v[r, sl]
                    return 0
                return lax.fori_loop(0, VECS_PER_ROW, vec_add, 0)

            lax.fori_loop(0, SCHUNK, row_add, 0)
            pltpu.sync_copy(rows_v, out_hbm.at[pl.ds(base, SCHUNK)])


def kernel(x, token_table):
    pos = _positional_table()
    x_flat = x.reshape(-1).astype(jnp.int32)
    mesh = plsc.VectorSubcoreMesh(core_axis_name="c", subcore_axis_name="s")
    out = pl.kernel(
        _sc_body,
        mesh=mesh,
        out_type=jax.ShapeDtypeStruct((BATCH * SEQ_LEN, DIM), jnp.float32),
        scratch_types=[
            pltpu.VMEM((SCHUNK,), jnp.int32),
            pltpu.VMEM((SCHUNK, DIM), jnp.float32),
            pltpu.VMEM((SCHUNK, DIM), jnp.float32),
            pltpu.SemaphoreType.DMA,
        ],
    )(token_table, x_flat, pos)
    return out.reshape(BATCH, SEQ_LEN, DIM)


# speedup vs baseline: 1.5674x; 1.5674x over previous
import jax
import jax.numpy as jnp
from jax import lax
from jax.experimental import pallas as pl
from jax.experimental.pallas import tpu as pltpu
from jax.experimental.pallas import tpu_sc as plsc

VOCAB = 100000
SEQ_LEN = 2048
DIM = 768
BATCH = 4
N = 10000

NUM_CORES = 2
NUM_SUBCORES = 16
NW = NUM_CORES * NUM_SUBCORES
SCHUNK = 32
NCHUNK = SEQ_LEN // SCHUNK
CHUNKS_PER_W = NCHUNK // NW
LANES = 16
VECS_PER_ROW = DIM // LANES
NSTEP = CHUNKS_PER_W * BATCH


def _positional_table():
    positions = jnp.arange(0, SEQ_LEN, dtype=jnp.float32)[:, None]
    den_even = jnp.power(float(N), 2.0 * jnp.arange(0, DIM, 2, dtype=jnp.float32) / DIM)
    den_odd = jnp.power(float(N), 2.0 * jnp.arange(1, DIM, 2, dtype=jnp.float32) / DIM)
    emb = jnp.zeros((SEQ_LEN, DIM), dtype=jnp.float32)
    emb = emb.at[:, 0::2].set(jnp.sin(positions / den_even))
    emb = emb.at[:, 1::2].set(jnp.cos(positions / den_odd))
    return emb


def _sc_body(table_hbm, x_hbm, pos_hbm, out_hbm,
             idx0, idx1, pos0, pos1, rows0, rows1,
             gsem0, gsem1, osem0, osem1):
    wid = lax.axis_index("s") * NUM_CORES + lax.axis_index("c")
    idx_v = (idx0, idx1)
    pos_v = (pos0, pos1)
    rows_v = (rows0, rows1)
    gsem = (gsem0, gsem1)
    osem = (osem0, osem1)

    def start_gather(i):
        slot = i % 2
        k, b = divmod(i, BATCH)
        s0 = (wid * CHUNKS_PER_W + k) * SCHUNK
        pltpu.sync_copy(x_hbm.at[pl.ds(b * SEQ_LEN + s0, SCHUNK)], idx_v[slot])
        return pltpu.async_copy(table_hbm.at[idx_v[slot]], rows_v[slot], gsem[slot])

    pltpu.sync_copy(pos_hbm.at[pl.ds(wid * CHUNKS_PER_W * SCHUNK, SCHUNK)], pos0)
    gathers = [start_gather(0), None]
    out_writes = [None, None]

    for i in range(NSTEP):
        slot = i % 2
        k, b = divmod(i, BATCH)
        if i + 1 < NSTEP:
            nslot = 1 - slot
            if out_writes[nslot] is not None:
                out_writes[nslot].wait()
                out_writes[nslot] = None
            gathers[nslot] = start_gather(i + 1)
            if (i + 1) % BATCH == 0:
                pltpu.sync_copy(
                    pos_hbm.at[pl.ds((wid * CHUNKS_PER_W + k + 1) * SCHUNK, SCHUNK)],
                    pos_v[(k + 1) % 2])
        gathers[slot].wait()
        pv = pos_v[k % 2] if CHUNKS_PER_W == 2 else pos_v[0]
        rv = rows_v[slot]

        def row_add(r, _, rv=rv, pv=pv):
            for j in range(VECS_PER_ROW):
                sl = pl.ds(j * LANES, LANES)
                rv[r, sl] = rv[r, sl] + pv[r, sl]
            return 0

        lax.fori_loop(0, SCHUNK, row_add, 0)
        if out_writes[slot] is not None:
            out_writes[slot].wait()
        s0 = (wid * CHUNKS_PER_W + k) * SCHUNK
        out_writes[slot] = pltpu.async_copy(
            rv, out_hbm.at[pl.ds(b * SEQ_LEN + s0, SCHUNK)], osem[slot])
    for w in out_writes:
        if w is not None:
            w.wait()


def kernel(x, token_table):
    pos = _positional_table()
    x_flat = x.reshape(-1).astype(jnp.int32)
    mesh = plsc.VectorSubcoreMesh(core_axis_name="c", subcore_axis_name="s")
    out = pl.kernel(
        _sc_body,
        mesh=mesh,
        out_type=jax.ShapeDtypeStruct((BATCH * SEQ_LEN, DIM), jnp.float32),
        scratch_types=[
            pltpu.VMEM((SCHUNK,), jnp.int32),
            pltpu.VMEM((SCHUNK,), jnp.int32),
            pltpu.VMEM((SCHUNK, DIM), jnp.float32),
            pltpu.VMEM((SCHUNK, DIM), jnp.float32),
            pltpu.VMEM((SCHUNK, DIM), jnp.float32),
            pltpu.VMEM((SCHUNK, DIM), jnp.float32),
            pltpu.SemaphoreType.DMA,
            pltpu.SemaphoreType.DMA,
            pltpu.SemaphoreType.DMA,
            pltpu.SemaphoreType.DMA,
        ],
    )(token_table, x_flat, pos)
    return out.reshape(BATCH, SEQ_LEN, DIM)


# speedup vs baseline: 2.4772x; 1.5805x over previous
import numpy as np

import jax
import jax.numpy as jnp
from jax import lax
from jax.experimental import pallas as pl
from jax.experimental.pallas import tpu as pltpu
from jax.experimental.pallas import tpu_sc as plsc

VOCAB = 100000
SEQ_LEN = 2048
DIM = 768
BATCH = 4
N = 10000

NUM_CORES = 2
NUM_SUBCORES = 16
NW = NUM_CORES * NUM_SUBCORES
SCHUNK = 32
NCHUNK = SEQ_LEN // SCHUNK
CHUNKS_PER_W = NCHUNK // NW
LANES = 16
VECS_PER_ROW = DIM // LANES
NSTEP = CHUNKS_PER_W * BATCH


def _positional_table():
    positions = np.arange(0, SEQ_LEN, dtype=np.float32)[:, None]
    den_even = np.power(float(N), 2.0 * np.arange(0, DIM, 2, dtype=np.float32) / DIM)
    den_odd = np.power(float(N), 2.0 * np.arange(1, DIM, 2, dtype=np.float32) / DIM)
    emb = np.zeros((SEQ_LEN, DIM), dtype=np.float32)
    emb[:, 0::2] = np.sin(positions / den_even)
    emb[:, 1::2] = np.cos(positions / den_odd)
    return jnp.asarray(emb)


def _sc_body(table_hbm, x_hbm, pos_hbm, out_hbm,
             idx0, idx1, pos0, pos1, rows0, rows1,
             gsem0, gsem1, osem0, osem1):
    wid = lax.axis_index("s") * NUM_CORES + lax.axis_index("c")
    idx_v = (idx0, idx1)
    pos_v = (pos0, pos1)
    rows_v = (rows0, rows1)
    gsem = (gsem0, gsem1)
    osem = (osem0, osem1)

    def start_gather(i):
        slot = i % 2
        k, b = divmod(i, BATCH)
        s0 = (wid * CHUNKS_PER_W + k) * SCHUNK
        pltpu.sync_copy(x_hbm.at[pl.ds(b * SEQ_LEN + s0, SCHUNK)], idx_v[slot])
        return pltpu.async_copy(table_hbm.at[idx_v[slot]], rows_v[slot], gsem[slot])

    pltpu.sync_copy(pos_hbm.at[pl.ds(wid * CHUNKS_PER_W * SCHUNK, SCHUNK)], pos0)
    gathers = [start_gather(0), None]
    out_writes = [None, None]

    for i in range(NSTEP):
        slot = i % 2
        k, b = divmod(i, BATCH)
        if i + 1 < NSTEP:
            nslot = 1 - slot
            if out_writes[nslot] is not None:
                out_writes[nslot].wait()
                out_writes[nslot] = None
            gathers[nslot] = start_gather(i + 1)
            if (i + 1) % BATCH == 0:
                pltpu.sync_copy(
                    pos_hbm.at[pl.ds((wid * CHUNKS_PER_W + k + 1) * SCHUNK, SCHUNK)],
                    pos_v[(k + 1) % 2])
        gathers[slot].wait()
        pv = pos_v[k % 2] if CHUNKS_PER_W == 2 else pos_v[0]
        rv = rows_v[slot]

        def row_add(r, _, rv=rv, pv=pv):
            for j in range(VECS_PER_ROW):
                sl = pl.ds(j * LANES, LANES)
                plsc.addupdate(rv.at[r, sl], pv[r, sl])
            return 0

        lax.fori_loop(0, SCHUNK, row_add, 0)
        if out_writes[slot] is not None:
            out_writes[slot].wait()
        s0 = (wid * CHUNKS_PER_W + k) * SCHUNK
        out_writes[slot] = pltpu.async_copy(
            rv, out_hbm.at[pl.ds(b * SEQ_LEN + s0, SCHUNK)], osem[slot])
    for w in out_writes:
        if w is not None:
            w.wait()


def kernel(x, token_table):
    pos = _positional_table()
    x_flat = x.reshape(-1).astype(jnp.int32)
    mesh = plsc.VectorSubcoreMesh(core_axis_name="c", subcore_axis_name="s")
    out = pl.kernel(
        _sc_body,
        mesh=mesh,
        out_type=jax.ShapeDtypeStruct((BATCH * SEQ_LEN, DIM), jnp.float32),
        scratch_types=[
            pltpu.VMEM((SCHUNK,), jnp.int32),
            pltpu.VMEM((SCHUNK,), jnp.int32),
            pltpu.VMEM((SCHUNK, DIM), jnp.float32),
            pltpu.VMEM((SCHUNK, DIM), jnp.float32),
            pltpu.VMEM((SCHUNK, DIM), jnp.float32),
            pltpu.VMEM((SCHUNK, DIM), jnp.float32),
            pltpu.SemaphoreType.DMA,
            pltpu.SemaphoreType.DMA,
            pltpu.SemaphoreType.DMA,
            pltpu.SemaphoreType.DMA,
        ],
    )(token_table, x_flat, pos)
    return out.reshape(BATCH, SEQ_LEN, DIM)


# speedup vs baseline: 2.7096x; 1.0938x over previous
import numpy as np

import jax
import jax.numpy as jnp
from jax import lax
from jax.experimental import pallas as pl
from jax.experimental.pallas import tpu as pltpu
from jax.experimental.pallas import tpu_sc as plsc

VOCAB = 100000
SEQ_LEN = 2048
DIM = 768
BATCH = 4
N = 10000

NUM_CORES = 2
NUM_SUBCORES = 16
NW = NUM_CORES * NUM_SUBCORES
SCHUNK = 32
NCHUNK = SEQ_LEN // SCHUNK
CHUNKS_PER_W = NCHUNK // NW
LANES = 16
VECS_PER_ROW = DIM // LANES
NSTEP = CHUNKS_PER_W * BATCH
NBUF = 3


def _positional_table():
    positions = np.arange(0, SEQ_LEN, dtype=np.float32)[:, None]
    den_even = np.power(float(N), 2.0 * np.arange(0, DIM, 2, dtype=np.float32) / DIM)
    den_odd = np.power(float(N), 2.0 * np.arange(1, DIM, 2, dtype=np.float32) / DIM)
    emb = np.zeros((SEQ_LEN, DIM), dtype=np.float32)
    emb[:, 0::2] = np.sin(positions / den_even)
    emb[:, 1::2] = np.cos(positions / den_odd)
    return jnp.asarray(emb)


def _step_addr(wid, i):
    k, b = divmod(i, BATCH)
    s0 = (wid * CHUNKS_PER_W + k) * SCHUNK
    return s0, b * SEQ_LEN + s0


def _sc_body(table_hbm, x_hbm, pos_hbm, out_hbm,
             idx_all, pos0, pos1, rows0, rows1, rows2,
             isem, ppsem, gsem0, gsem1, gsem2, osem0, osem1, osem2):
    wid = lax.axis_index("s") * NUM_CORES + lax.axis_index("c")
    pos_v = (pos0, pos1)
    rows_v = (rows0, rows1, rows2)
    gsem = (gsem0, gsem1, gsem2)
    osem = (osem0, osem1, osem2)

    cps = []
    for i in range(NSTEP):
        s0, base = _step_addr(wid, i)
        cps.append(pltpu.async_copy(
            x_hbm.at[pl.ds(base, SCHUNK)], idx_all.at[i], isem))
    for k in range(CHUNKS_PER_W):
        s0 = (wid * CHUNKS_PER_W + k) * SCHUNK
        cps.append(pltpu.async_copy(
            pos_hbm.at[pl.ds(s0, SCHUNK)], pos_v[k], ppsem))
    for cp in cps:
        cp.wait()

    def fire_gather(i):
        slot = i % NBUF
        return pltpu.async_copy(
            table_hbm.at[idx_all.at[i]], rows_v[slot], gsem[slot])

    gathers = [None] * NBUF
    out_writes = [None] * NBUF
    gathers[0] = fire_gather(0)
    gathers[1] = fire_gather(1)

    for i in range(NSTEP):
        slot = i % NBUF
        k = i // BATCH
        if i + 2 < NSTEP:
            nslot = (i + 2) % NBUF
            if out_writes[nslot] is not None:
                out_writes[nslot].wait()
                out_writes[nslot] = None
            gathers[nslot] = fire_gather(i + 2)
        gathers[slot].wait()
        rv = rows_v[slot]
        pv = pos_v[k]

        def row_add(r, _, rv=rv, pv=pv):
            for j in range(VECS_PER_ROW):
                sl = pl.ds(j * LANES, LANES)
                plsc.addupdate(rv.at[r, sl], pv[r, sl])
            return 0

        lax.fori_loop(0, SCHUNK, row_add, 0)
        _, base = _step_addr(wid, i)
        out_writes[slot] = pltpu.async_copy(
            rv, out_hbm.at[pl.ds(base, SCHUNK)], osem[slot])
    for w in out_writes:
        if w is not None:
            w.wait()


def kernel(x, token_table):
    pos = _positional_table()
    x_flat = x.reshape(-1).astype(jnp.int32)
    mesh = plsc.VectorSubcoreMesh(core_axis_name="c", subcore_axis_name="s")
    out = pl.kernel(
        _sc_body,
        mesh=mesh,
        out_type=jax.ShapeDtypeStruct((BATCH * SEQ_LEN, DIM), jnp.float32),
        scratch_types=[
            pltpu.VMEM((NSTEP, SCHUNK), jnp.int32),
            pltpu.VMEM((SCHUNK, DIM), jnp.float32),
            pltpu.VMEM((SCHUNK, DIM), jnp.float32),
            pltpu.VMEM((SCHUNK, DIM), jnp.float32),
            pltpu.VMEM((SCHUNK, DIM), jnp.float32),
            pltpu.VMEM((SCHUNK, DIM), jnp.float32),
        ] + [pltpu.SemaphoreType.DMA] * 8,
    )(token_table, x_flat, pos)
    return out.reshape(BATCH, SEQ_LEN, DIM)
